# Initial kernel scaffold; baseline (speedup 1.0000x reference)
import functools

import jax
import jax.numpy as jnp
from jax import lax
from jax.experimental import pallas as pl
from jax.experimental.pallas import tpu as pltpu
from jax.experimental.pallas import tpu_sc as plsc

N_NODES = 10000
D_FEAT = 128
N_EDGES = 320000
NUM_ITERATIONS = 10

NUM_CORES = 2
NUM_SUBCORES = 16
NUM_TILES = NUM_CORES * NUM_SUBCORES
EDGE_BLOCK = 128

_E_PER_TILE = -(-N_EDGES // NUM_TILES)
_BLOCKS_PER_TILE = -(-_E_PER_TILE // EDGE_BLOCK)
_E_PER_TILE_PAD = _BLOCKS_PER_TILE * EDGE_BLOCK
_ROWS_PER_TILE = N_NODES // NUM_SUBCORES


def _propagate_step(cur, x0_half, srcs, dsts, ws):
    mesh = plsc.VectorSubcoreMesh(core_axis_name="c", subcore_axis_name="s")

    @functools.partial(
        pl.kernel,
        out_type=(
            jax.ShapeDtypeStruct((N_NODES, D_FEAT), jnp.float32),
            jax.ShapeDtypeStruct((N_NODES, D_FEAT), jnp.float32),
        ),
        mesh=mesh,
        scratch_types=[
            pltpu.VMEM_SHARED((N_NODES, D_FEAT), jnp.float32),
            pltpu.VMEM((EDGE_BLOCK,), jnp.int32),
            pltpu.VMEM((EDGE_BLOCK,), jnp.int32),
            pltpu.VMEM((EDGE_BLOCK,), jnp.float32),
            pltpu.VMEM((EDGE_BLOCK, D_FEAT), jnp.float32),
            pltpu.SemaphoreType.DMA,
        ],
    )
    def step(cur_hbm, x0_hbm, src_hbm, dst_hbm, w_hbm, outa_hbm, outb_hbm,
             acc_sh, sidx_v, didx_v, w_v, rows_v, sem):
        c = lax.axis_index("c")
        s = lax.axis_index("s")
        wid = c * NUM_SUBCORES + s

        row0 = s * _ROWS_PER_TILE
        pltpu.sync_copy(x0_hbm.at[pl.ds(row0, _ROWS_PER_TILE)],
                        acc_sh.at[pl.ds(row0, _ROWS_PER_TILE)])
        plsc.subcore_barrier()

        ebase = wid * _E_PER_TILE_PAD

        @pl.loop(0, _BLOCKS_PER_TILE)
        def _(b):
            base = ebase + b * EDGE_BLOCK
            pltpu.sync_copy(src_hbm.at[pl.ds(base, EDGE_BLOCK)], sidx_v)
            pltpu.sync_copy(dst_hbm.at[pl.ds(base, EDGE_BLOCK)], didx_v)
            pltpu.sync_copy(w_hbm.at[pl.ds(base, EDGE_BLOCK)], w_v)
            pltpu.async_copy(cur_hbm.at[sidx_v], rows_v, sem).wait()

            @pl.loop(0, EDGE_BLOCK)
            def _(e):
                wv = plsc.load_gather(w_v, [jnp.full((16,), e, jnp.int32)])
                for j in range(D_FEAT // 16):
                    sl = pl.ds(j * 16, 16)
                    rows_v[e, sl] = rows_v[e, sl] * wv

            pltpu.sync_copy(rows_v, acc_sh.at[didx_v], add=True)

        plsc.subcore_barrier()

        @pl.when(c == 0)
        def _():
            pltpu.sync_copy(acc_sh.at[pl.ds(row0, _ROWS_PER_TILE)],
                            outa_hbm.at[pl.ds(row0, _ROWS_PER_TILE)])

        @pl.when(c == 1)
        def _():
            pltpu.sync_copy(acc_sh.at[pl.ds(row0, _ROWS_PER_TILE)],
                            outb_hbm.at[pl.ds(row0, _ROWS_PER_TILE)])

    return step(cur, x0_half, srcs, dsts, ws)


def _pad_per_tile(a, pad_value):
    a = a.reshape(NUM_TILES, _E_PER_TILE)
    a = jnp.pad(a, ((0, 0), (0, _E_PER_TILE_PAD - _E_PER_TILE)),
                constant_values=pad_value)
    return a.reshape(NUM_TILES * _E_PER_TILE_PAD)


def kernel(x, mask, edge_index, edge_weight):
    src = edge_index[0].astype(jnp.int32)
    dst = edge_index[1].astype(jnp.int32)
    w = jnp.where(mask[dst], 0.0, edge_weight.astype(jnp.float32))
    x0_half = jnp.where(mask[:, None], x, 0.0) * 0.5

    srcs = _pad_per_tile(src, 0)
    dsts = _pad_per_tile(dst, 0)
    ws = _pad_per_tile(w, 0.0)

    cur = x0_half * 2.0
    for _ in range(NUM_ITERATIONS):
        pa, pb = _propagate_step(cur, x0_half, srcs, dsts, ws)
        cur = pa + pb
    return cur


# speedup vs baseline: 2.2109x; 2.2109x over previous
import dataclasses
import functools

import jax
import jax.numpy as jnp
from jax import lax
from jax.experimental import pallas as pl
from jax.experimental.pallas import tpu as pltpu
from jax.experimental.pallas import tpu_sc as plsc

N_NODES = 10000
D_FEAT = 128
N_EDGES = 320000
NUM_ITERATIONS = 10

NUM_CORES = 2
NUM_SUBCORES = 16
NUM_TILES = NUM_CORES * NUM_SUBCORES
EDGE_BLOCK = 128

_E_PER_TILE = -(-N_EDGES // NUM_TILES)
_BLOCKS_PER_TILE = -(-_E_PER_TILE // EDGE_BLOCK)
_E_PER_TILE_PAD = _BLOCKS_PER_TILE * EDGE_BLOCK
_ROWS_PER_TILE = 8 * (-(-N_NODES // (8 * NUM_SUBCORES)))
N_PAD = _ROWS_PER_TILE * NUM_SUBCORES


def _propagate_step(cur, x0_half, srcs, dsts, ws):
    mesh = plsc.VectorSubcoreMesh(core_axis_name="c", subcore_axis_name="s")
    cp = pltpu.CompilerParams()
    if "needs_layout_passes" in pltpu.CompilerParams.__dataclass_fields__:
        cp = dataclasses.replace(cp, needs_layout_passes=False)

    @functools.partial(
        pl.kernel,
        compiler_params=cp,
        out_type=(
            jax.ShapeDtypeStruct((N_PAD, D_FEAT), jnp.float32),
            jax.ShapeDtypeStruct((N_PAD, D_FEAT), jnp.float32),
        ),
        mesh=mesh,
        scratch_types=[
            pltpu.VMEM_SHARED((N_PAD, D_FEAT), jnp.float32),
            pltpu.VMEM((EDGE_BLOCK,), jnp.int32),
            pltpu.VMEM((EDGE_BLOCK,), jnp.int32),
            pltpu.VMEM((EDGE_BLOCK,), jnp.float32),
            pltpu.VMEM((EDGE_BLOCK, D_FEAT), jnp.float32),
            pltpu.SemaphoreType.DMA,
        ],
    )
    def step(cur_hbm, x0_hbm, src_hbm, dst_hbm, w_hbm, outa_hbm, outb_hbm,
             acc_sh, sidx_v, didx_v, w_v, rows_v, sem):
        c = lax.axis_index("c")
        s = lax.axis_index("s")
        wid = c * NUM_SUBCORES + s

        row0 = s * _ROWS_PER_TILE
        pltpu.sync_copy(x0_hbm.at[pl.ds(row0, _ROWS_PER_TILE)],
                        acc_sh.at[pl.ds(row0, _ROWS_PER_TILE)])
        plsc.subcore_barrier()

        ebase = wid * _E_PER_TILE_PAD

        @pl.loop(0, _BLOCKS_PER_TILE)
        def _(b):
            base = ebase + b * EDGE_BLOCK
            pltpu.sync_copy(src_hbm.at[pl.ds(base, EDGE_BLOCK)], sidx_v)
            pltpu.sync_copy(dst_hbm.at[pl.ds(base, EDGE_BLOCK)], didx_v)
            pltpu.sync_copy(w_hbm.at[pl.ds(base, EDGE_BLOCK)], w_v)
            pltpu.async_copy(cur_hbm.at[sidx_v], rows_v, sem).wait()

            @pl.loop(0, EDGE_BLOCK)
            def _(e):
                wv = plsc.load_gather(w_v, [jnp.full((16,), e, jnp.int32)])
                for j in range(D_FEAT // 16):
                    sl = pl.ds(j * 16, 16)
                    rows_v[e, sl] = rows_v[e, sl] * wv

            pltpu.sync_copy(rows_v, acc_sh.at[didx_v], add=True)

        plsc.subcore_barrier()

        @pl.when(c == 0)
        def _():
            pltpu.sync_copy(acc_sh.at[pl.ds(row0, _ROWS_PER_TILE)],
                            outa_hbm.at[pl.ds(row0, _ROWS_PER_TILE)])

        @pl.when(c == 1)
        def _():
            pltpu.sync_copy(acc_sh.at[pl.ds(row0, _ROWS_PER_TILE)],
                            outb_hbm.at[pl.ds(row0, _ROWS_PER_TILE)])

    return step(cur, x0_half, srcs, dsts, ws)


def _pad_per_tile(a, pad_value):
    a = a.reshape(NUM_TILES, _E_PER_TILE)
    a = jnp.pad(a, ((0, 0), (0, _E_PER_TILE_PAD - _E_PER_TILE)),
                constant_values=pad_value)
    return a.reshape(NUM_TILES * _E_PER_TILE_PAD)


def kernel(x, mask, edge_index, edge_weight):
    src = edge_index[0].astype(jnp.int32)
    dst = edge_index[1].astype(jnp.int32)
    w = jnp.where(mask[dst], 0.0, edge_weight.astype(jnp.float32))
    x0_half = jnp.where(mask[:, None], x, 0.0) * 0.5
    x0_half = jnp.pad(x0_half, ((0, N_PAD - N_NODES), (0, 0)))

    srcs = _pad_per_tile(src, 0)
    dsts = _pad_per_tile(dst, 0)
    ws = _pad_per_tile(w, 0.0)

    cur = x0_half * 2.0
    for _ in range(NUM_ITERATIONS):
        pa, pb = _propagate_step(cur, x0_half, srcs, dsts, ws)
        cur = pa + pb
    return cur[:N_NODES]


# speedup vs baseline: 2.3438x; 1.0601x over previous
import dataclasses
import functools

import jax
import jax.numpy as jnp
from jax import lax
from jax.experimental import pallas as pl
from jax.experimental.pallas import tpu as pltpu
from jax.experimental.pallas import tpu_sc as plsc

N_NODES = 10000
D_FEAT = 128
N_EDGES = 320000
NUM_ITERATIONS = 10

NUM_CORES = 2
NUM_SUBCORES = 16
NUM_TILES = NUM_CORES * NUM_SUBCORES
EDGE_BLOCK = 128
NRING = 4

_E_PER_TILE = -(-N_EDGES // NUM_TILES)
_BLOCKS_PER_TILE = NRING * (-(-_E_PER_TILE // (EDGE_BLOCK * NRING)))
_E_PER_TILE_PAD = _BLOCKS_PER_TILE * EDGE_BLOCK
_TOTAL_BLOCKS = NUM_TILES * _BLOCKS_PER_TILE
_ROWS_PER_TILE = 8 * (-(-N_NODES // (8 * NUM_SUBCORES)))
N_PAD = _ROWS_PER_TILE * NUM_SUBCORES


def _propagate_step(cur, x0_half, eblk):
    mesh = plsc.VectorSubcoreMesh(core_axis_name="c", subcore_axis_name="s")
    cp = pltpu.CompilerParams()
    if "needs_layout_passes" in pltpu.CompilerParams.__dataclass_fields__:
        cp = dataclasses.replace(cp, needs_layout_passes=False)

    @functools.partial(
        pl.kernel,
        compiler_params=cp,
        out_type=(
            jax.ShapeDtypeStruct((N_PAD, D_FEAT), jnp.float32),
            jax.ShapeDtypeStruct((N_PAD, D_FEAT), jnp.float32),
        ),
        mesh=mesh,
        scratch_types=[
            pltpu.VMEM_SHARED((N_PAD, D_FEAT), jnp.float32),
            pltpu.VMEM((NRING, 3, EDGE_BLOCK), jnp.int32),
            pltpu.VMEM((2, EDGE_BLOCK, D_FEAT), jnp.float32),
            pltpu.SemaphoreType.DMA((NRING,)),
            pltpu.SemaphoreType.DMA((2,)),
            pltpu.SemaphoreType.DMA((2,)),
        ],
    )
    def step(cur_hbm, x0_hbm, eblk_hbm, outa_hbm, outb_hbm,
             acc_sh, edges_v, rows_v, semE, semG, semS):
        c = lax.axis_index("c")
        s = lax.axis_index("s")
        wid = c * NUM_SUBCORES + s
        B = _BLOCKS_PER_TILE
        blk0 = wid * B

        row0 = s * _ROWS_PER_TILE
        pltpu.sync_copy(x0_hbm.at[pl.ds(row0, _ROWS_PER_TILE)],
                        acc_sh.at[pl.ds(row0, _ROWS_PER_TILE)])
        plsc.subcore_barrier()

        def e_load(i, p):
            return pltpu.async_copy(eblk_hbm.at[blk0 + i], edges_v.at[p],
                                    semE.at[p])

        def g_copy(pe, pr):
            return pltpu.make_async_copy(cur_hbm.at[edges_v.at[pe, 0]],
                                         rows_v.at[pr], semG.at[pr])

        def s_copy(pe, pr):
            return pltpu.make_async_copy(rows_v.at[pr],
                                         acc_sh.at[edges_v.at[pe, 1]],
                                         semS.at[pr])

        def multiply(pe, pr):
            @pl.loop(0, EDGE_BLOCK)
            def _(e):
                wvi = plsc.load_gather(edges_v.at[pe, 2],
                                       [jnp.full((16,), e, jnp.int32)])
                wv = plsc.bitcast(wvi, jnp.float32)
                for j in range(D_FEAT // 16):
                    sl = pl.ds(j * 16, 16)
                    rows_v[pr, e, sl] = rows_v[pr, e, sl] * wv

        e_load(0, 0)
        e_load(1, 1)
        pltpu.make_async_copy(eblk_hbm.at[blk0], edges_v.at[0],
                              semE.at[0]).wait()
        g_copy(0, 0).start()

        @pl.loop(0, B, step=NRING)
        def _(i0):
            for p in range(NRING):
                i = i0 + p
                pr = p % 2

                @pl.when(i >= 1)
                def _():
                    s_copy((p - 1) % NRING, (pr + 1) % 2).wait()

                @pl.when(i + 2 < B)
                def _():
                    e_load(i + 2, (p + 2) % NRING)

                @pl.when(i + 1 < B)
                def _():
                    pltpu.make_async_copy(eblk_hbm.at[blk0 + i + 1],
                                          edges_v.at[(p + 1) % NRING],
                                          semE.at[(p + 1) % NRING]).wait()
                    g_copy((p + 1) % NRING, (pr + 1) % 2).start()

                g_copy(p, pr).wait()
                multiply(p, pr)
                s_copy(p, pr).start(add=True)

        s_copy((B - 1) % NRING, (B - 1) % 2).wait()

        plsc.subcore_barrier()

        @pl.when(c == 0)
        def _():
            pltpu.sync_copy(acc_sh.at[pl.ds(row0, _ROWS_PER_TILE)],
                            outa_hbm.at[pl.ds(row0, _ROWS_PER_TILE)])

        @pl.when(c == 1)
        def _():
            pltpu.sync_copy(acc_sh.at[pl.ds(row0, _ROWS_PER_TILE)],
                            outb_hbm.at[pl.ds(row0, _ROWS_PER_TILE)])

    return step(cur, x0_half, eblk)


def _pad_per_tile(a, pad_value):
    a = a.reshape(NUM_TILES, _E_PER_TILE)
    a = jnp.pad(a, ((0, 0), (0, _E_PER_TILE_PAD - _E_PER_TILE)),
                constant_values=pad_value)
    return a.reshape(NUM_TILES * _E_PER_TILE_PAD)


def kernel(x, mask, edge_index, edge_weight):
    src = edge_index[0].astype(jnp.int32)
    dst = edge_index[1].astype(jnp.int32)
    w = jnp.where(mask[dst], 0.0, edge_weight.astype(jnp.float32))
    x0_half = jnp.where(mask[:, None], x, 0.0) * 0.5
    x0_half = jnp.pad(x0_half, ((0, N_PAD - N_NODES), (0, 0)))

    srcs = _pad_per_tile(src, 0).reshape(_TOTAL_BLOCKS, EDGE_BLOCK)
    dsts = _pad_per_tile(dst, 0).reshape(_TOTAL_BLOCKS, EDGE_BLOCK)
    wbits = lax.bitcast_convert_type(_pad_per_tile(w, 0.0),
                                     jnp.int32).reshape(_TOTAL_BLOCKS,
                                                        EDGE_BLOCK)
    eblk = jnp.stack([srcs, dsts, wbits], axis=1)

    cur = x0_half * 2.0
    for _ in range(NUM_ITERATIONS):
        pa, pb = _propagate_step(cur, x0_half, eblk)
        cur = pa + pb
    return cur[:N_NODES]


# speedup vs baseline: 3.6455x; 1.5554x over previous
import dataclasses
import functools

import jax
import jax.numpy as jnp
from jax import lax
from jax.experimental import pallas as pl
from jax.experimental.pallas import tpu as pltpu
from jax.experimental.pallas import tpu_sc as plsc

N_NODES = 10000
D_FEAT = 128
N_EDGES = 320000
NUM_ITERATIONS = 10

NUM_CORES = 2
NUM_SUBCORES = 16
HALF_F = D_FEAT // NUM_CORES
EDGE_BLOCK = 128
NRING = 4

_E_PER_TILE = -(-N_EDGES // NUM_SUBCORES)
_BLOCKS_PER_TILE = NRING * (-(-_E_PER_TILE // (EDGE_BLOCK * NRING)))
_E_PER_TILE_PAD = _BLOCKS_PER_TILE * EDGE_BLOCK
_TOTAL_BLOCKS = NUM_SUBCORES * _BLOCKS_PER_TILE
_ROWS_PER_TILE = 8 * (-(-N_NODES // (8 * NUM_SUBCORES)))
N_PAD = _ROWS_PER_TILE * NUM_SUBCORES


def _propagate(x0s, eblk):
    mesh = plsc.VectorSubcoreMesh(core_axis_name="c", subcore_axis_name="s")
    cp = pltpu.CompilerParams()
    if "needs_layout_passes" in pltpu.CompilerParams.__dataclass_fields__:
        cp = dataclasses.replace(cp, needs_layout_passes=False)
    if "use_tc_tiling_on_sc" in pltpu.CompilerParams.__dataclass_fields__:
        cp = dataclasses.replace(cp, use_tc_tiling_on_sc=False)

    @functools.partial(
        pl.kernel,
        compiler_params=cp,
        out_type=(
            jax.ShapeDtypeStruct((N_PAD, HALF_F), jnp.float32),
            jax.ShapeDtypeStruct((N_PAD, HALF_F), jnp.float32),
        ),
        mesh=mesh,
        scratch_types=[
            pltpu.VMEM_SHARED((N_PAD, HALF_F), jnp.float32),
            pltpu.VMEM_SHARED((N_PAD, HALF_F), jnp.float32),
            pltpu.VMEM((NRING, 3, EDGE_BLOCK), jnp.int32),
            pltpu.VMEM((2, EDGE_BLOCK, HALF_F), jnp.float32),
            pltpu.SemaphoreType.DMA((NRING,)),
            pltpu.SemaphoreType.DMA((2,)),
            pltpu.SemaphoreType.DMA((2,)),
        ],
    )
    def prop(x0s_hbm, eblk_hbm, outa_hbm, outb_hbm,
             buf_a, buf_b, edges_v, rows_v, semE, semG, semS):
        c = lax.axis_index("c")
        s = lax.axis_index("s")
        B = _BLOCKS_PER_TILE
        blk0 = s * B
        row0 = s * _ROWS_PER_TILE
        rsl = pl.ds(row0, _ROWS_PER_TILE)

        def seed(wbuf):
            pltpu.sync_copy(x0s_hbm.at[c].at[rsl], wbuf.at[rsl])

        def edge_pass(rbuf, wbuf):
            def e_load(i, p):
                return pltpu.async_copy(eblk_hbm.at[blk0 + i], edges_v.at[p],
                                        semE.at[p])

            def g_copy(pe, pr):
                return pltpu.make_async_copy(rbuf.at[edges_v.at[pe, 0]],
                                             rows_v.at[pr], semG.at[pr])

            def s_copy(pe, pr):
                return pltpu.make_async_copy(rows_v.at[pr],
                                             wbuf.at[edges_v.at[pe, 1]],
                                             semS.at[pr])

            def multiply(pe, pr):
                @pl.loop(0, EDGE_BLOCK)
                def _(e):
                    wvi = plsc.load_gather(edges_v.at[pe, 2],
                                           [jnp.full((16,), e, jnp.int32)])
                    wv = plsc.bitcast(wvi, jnp.float32)
                    for j in range(HALF_F // 16):
                        sl = pl.ds(j * 16, 16)
                        rows_v[pr, e, sl] = rows_v[pr, e, sl] * wv

            e_load(0, 0)
            e_load(1, 1)
            pltpu.make_async_copy(eblk_hbm.at[blk0], edges_v.at[0],
                                  semE.at[0]).wait()
            g_copy(0, 0).start()

            @pl.loop(0, B, step=NRING)
            def _(i0):
                for p in range(NRING):
                    i = i0 + p
                    pr = p % 2

                    @pl.when(i >= 1)
                    def _():
                        s_copy((p - 1) % NRING, (pr + 1) % 2).wait()

                    @pl.when(i + 2 < B)
                    def _():
                        e_load(i + 2, (p + 2) % NRING)

                    @pl.when(i + 1 < B)
                    def _():
                        pltpu.make_async_copy(eblk_hbm.at[blk0 + i + 1],
                                              edges_v.at[(p + 1) % NRING],
                                              semE.at[(p + 1) % NRING]).wait()
                        g_copy((p + 1) % NRING, (pr + 1) % 2).start()

                    g_copy(p, pr).wait()
                    multiply(p, pr)
                    s_copy(p, pr).start(add=True)

            s_copy((B - 1) % NRING, (B - 1) % 2).wait()

        seed(buf_a)
        plsc.subcore_barrier()

        @pl.loop(0, NUM_ITERATIONS // 2)
        def _(t):
            for rbuf, wbuf in ((buf_a, buf_b), (buf_b, buf_a)):
                seed(wbuf)
                plsc.subcore_barrier()
                edge_pass(rbuf, wbuf)
                plsc.subcore_barrier()

        @pl.when(c == 0)
        def _():
            pltpu.sync_copy(buf_a.at[rsl], outa_hbm.at[rsl])

        @pl.when(c == 1)
        def _():
            pltpu.sync_copy(buf_a.at[rsl], outb_hbm.at[rsl])

    return prop(x0s, eblk)


def _pad_per_tile(a, pad_value):
    a = a.reshape(NUM_SUBCORES, _E_PER_TILE)
    a = jnp.pad(a, ((0, 0), (0, _E_PER_TILE_PAD - _E_PER_TILE)),
                constant_values=pad_value)
    return a.reshape(NUM_SUBCORES * _E_PER_TILE_PAD)


def kernel(x, mask, edge_index, edge_weight):
    src = edge_index[0].astype(jnp.int32)
    dst = edge_index[1].astype(jnp.int32)
    w = jnp.where(mask[dst], 0.0, edge_weight.astype(jnp.float32))
    x0 = jnp.where(mask[:, None], x, 0.0)
    x0 = jnp.pad(x0, ((0, N_PAD - N_NODES), (0, 0)))
    x0s = x0.reshape(N_PAD, NUM_CORES, HALF_F).transpose(1, 0, 2)

    srcs = _pad_per_tile(src, 0).reshape(_TOTAL_BLOCKS, EDGE_BLOCK)
    dsts = _pad_per_tile(dst, 0).reshape(_TOTAL_BLOCKS, EDGE_BLOCK)
    wbits = lax.bitcast_convert_type(_pad_per_tile(w, 0.0),
                                     jnp.int32).reshape(_TOTAL_BLOCKS,
                                                        EDGE_BLOCK)
    eblk = jnp.stack([srcs, dsts, wbits], axis=1)

    outa, outb = _propagate(x0s, eblk)
    out = jnp.stack([outa, outb], axis=1).reshape(N_PAD, D_FEAT)
    return out[:N_NODES]


# speedup vs baseline: 10.9347x; 2.9995x over previous
import dataclasses
import functools

import jax
import jax.numpy as jnp
from jax import lax
from jax.experimental import pallas as pl
from jax.experimental.pallas import tpu as pltpu
from jax.experimental.pallas import tpu_sc as plsc

N_NODES = 10000
D_FEAT = 128
N_EDGES = 320000
NUM_ITERATIONS = 10

NUM_CORES = 2
NUM_SUBCORES = 16
HALF_F = D_FEAT // NUM_CORES
EDGE_BLOCK = 128
NRING = 4

_E_PER_TILE = -(-N_EDGES // NUM_SUBCORES)
_BLOCKS_PER_TILE = NRING * (-(-_E_PER_TILE // (EDGE_BLOCK * NRING)))
_E_PER_TILE_PAD = _BLOCKS_PER_TILE * EDGE_BLOCK
_TOTAL_BLOCKS = NUM_SUBCORES * _BLOCKS_PER_TILE
_ROWS_PER_TILE = 8 * (-(-N_NODES // (8 * NUM_SUBCORES)))
N_PAD = _ROWS_PER_TILE * NUM_SUBCORES


def _compiler_params():
    cp = pltpu.CompilerParams()
    if "needs_layout_passes" in pltpu.CompilerParams.__dataclass_fields__:
        cp = dataclasses.replace(cp, needs_layout_passes=False)
    if "use_tc_tiling_on_sc" in pltpu.CompilerParams.__dataclass_fields__:
        cp = dataclasses.replace(cp, use_tc_tiling_on_sc=False)
    return cp


def _compact(eblk, maski):
    mesh = plsc.VectorSubcoreMesh(core_axis_name="c", subcore_axis_name="s")
    half_blocks = _BLOCKS_PER_TILE // 2

    @functools.partial(
        pl.kernel,
        compiler_params=_compiler_params(),
        out_type=(
            jax.ShapeDtypeStruct((NUM_SUBCORES, 3, _E_PER_TILE_PAD),
                                 jnp.int32),
            jax.ShapeDtypeStruct((NUM_SUBCORES, 16), jnp.int32),
        ),
        mesh=mesh,
        scratch_types=[
            pltpu.VMEM((N_PAD,), jnp.int32),
            pltpu.VMEM((half_blocks, 3, EDGE_BLOCK), jnp.int32),
            pltpu.VMEM((3, _E_PER_TILE_PAD), jnp.int32),
            pltpu.VMEM((16,), jnp.int32),
        ],
    )
    def comp(eblk_hbm, maski_hbm, out_hbm, cnt_hbm,
             mask_v, in_v, out_v, cw_v):
        c = lax.axis_index("c")
        s = lax.axis_index("s")

        @pl.when(c == 0)
        def _():
            pltpu.sync_copy(maski_hbm, mask_v)

            @pl.loop(0, _E_PER_TILE_PAD, step=16)
            def _(o):
                z = jnp.zeros((16,), jnp.int32)
                sl = pl.ds(o, 16)
                out_v[0, sl] = z
                out_v[1, sl] = z
                out_v[2, sl] = z

            blk0 = s * _BLOCKS_PER_TILE
            cnt = jnp.int32(0)
            for half in range(2):
                pltpu.sync_copy(
                    eblk_hbm.at[pl.ds(blk0 + half * half_blocks,
                                      half_blocks)], in_v)

                @pl.loop(0, half_blocks * (EDGE_BLOCK // 16),
                         init_carry=cnt)
                def cnt_loop(g, cnt):
                    b = g // (EDGE_BLOCK // 16)
                    sl = pl.ds((g % (EDGE_BLOCK // 16)) * 16, 16)
                    s16 = in_v[b, 0, sl]
                    d16 = in_v[b, 1, sl]
                    w16 = in_v[b, 2, sl]
                    keep = plsc.load_gather(mask_v, [d16]) == 0
                    osl = pl.ds(cnt, 16)
                    plsc.store_compressed(out_v.at[0, osl], s16, mask=keep)
                    plsc.store_compressed(out_v.at[1, osl], d16, mask=keep)
                    plsc.store_compressed(out_v.at[2, osl], w16, mask=keep)
                    return cnt + jnp.sum(jnp.where(keep, 1, 0))

                cnt = cnt_loop

            blocks = (cnt + EDGE_BLOCK - 1) // EDGE_BLOCK
            nblk = ((blocks + NRING - 1) // NRING) * NRING
            cw_v[...] = jnp.full((16,), nblk, jnp.int32)
            pltpu.sync_copy(out_v, out_hbm.at[s])
            pltpu.sync_copy(cw_v, cnt_hbm.at[s])

    return comp(eblk, maski)


def _propagate(x0s, eb2, cnt):
    mesh = plsc.VectorSubcoreMesh(core_axis_name="c", subcore_axis_name="s")

    @functools.partial(
        pl.kernel,
        compiler_params=_compiler_params(),
        out_type=(
            jax.ShapeDtypeStruct((N_PAD, HALF_F), jnp.float32),
            jax.ShapeDtypeStruct((N_PAD, HALF_F), jnp.float32),
        ),
        mesh=mesh,
        scratch_types=[
            pltpu.VMEM_SHARED((N_PAD, HALF_F), jnp.float32),
            pltpu.VMEM_SHARED((N_PAD, HALF_F), jnp.float32),
            pltpu.VMEM((NRING, 3, EDGE_BLOCK), jnp.int32),
            pltpu.VMEM((2, EDGE_BLOCK, HALF_F), jnp.float32),
            pltpu.VMEM((16,), jnp.int32),
            pltpu.SemaphoreType.DMA((NRING,)),
            pltpu.SemaphoreType.DMA((2,)),
            pltpu.SemaphoreType.DMA((2,)),
        ],
    )
    def prop(x0s_hbm, eb2_hbm, cnt_hbm, outa_hbm, outb_hbm,
             buf_a, buf_b, edges_v, rows_v, cnt_s, semE, semG, semS):
        c = lax.axis_index("c")
        s = lax.axis_index("s")
        row0 = s * _ROWS_PER_TILE
        rsl = pl.ds(row0, _ROWS_PER_TILE)

        pltpu.sync_copy(cnt_hbm.at[s], cnt_s)
        nblk = jnp.sum(jnp.where(lax.iota(jnp.int32, 16) == 0,
                                 cnt_s[...], 0), axis=0)

        def seed(wbuf):
            pltpu.sync_copy(x0s_hbm.at[c].at[rsl], wbuf.at[rsl])

        def edge_pass(rbuf, wbuf):
            def e_copy(i, p):
                return pltpu.make_async_copy(
                    eb2_hbm.at[s].at[:, pl.ds(i * EDGE_BLOCK, EDGE_BLOCK)],
                    edges_v.at[p], semE.at[p])

            def g_copy(pe, pr):
                return pltpu.make_async_copy(rbuf.at[edges_v.at[pe, 0]],
                                             rows_v.at[pr], semG.at[pr])

            def s_copy(pe, pr):
                return pltpu.make_async_copy(rows_v.at[pr],
                                             wbuf.at[edges_v.at[pe, 1]],
                                             semS.at[pr])

            def multiply(pe, pr):
                @pl.loop(0, EDGE_BLOCK)
                def _(e):
                    wvi = plsc.load_gather(edges_v.at[pe, 2],
                                           [jnp.full((16,), e, jnp.int32)])
                    wv = plsc.bitcast(wvi, jnp.float32)
                    for j in range(HALF_F // 16):
                        sl = pl.ds(j * 16, 16)
                        rows_v[pr, e, sl] = rows_v[pr, e, sl] * wv

            @pl.when(nblk > 0)
            def _():
                e_copy(0, 0).start()
                e_copy(1, 1).start()
                e_copy(0, 0).wait()
                g_copy(0, 0).start()

                @pl.loop(0, nblk, step=NRING)
                def _(i0):
                    for p in range(NRING):
                        i = i0 + p
                        pr = p % 2

                        @pl.when(i >= 1)
                        def _():
                            s_copy((p - 1) % NRING, (pr + 1) % 2).wait()

                        @pl.when(i + 2 < nblk)
                        def _():
                            e_copy(i + 2, (p + 2) % NRING).start()

                        @pl.when(i + 1 < nblk)
                        def _():
                            e_copy(i + 1, (p + 1) % NRING).wait()
                            g_copy((p + 1) % NRING, (pr + 1) % 2).start()

                        g_copy(p, pr).wait()
                        multiply(p, pr)
                        s_copy(p, pr).start(add=True)

                s_copy(NRING - 1, 1).wait()

        seed(buf_a)
        plsc.subcore_barrier()

        @pl.loop(0, NUM_ITERATIONS // 2)
        def _(t):
            for rbuf, wbuf in ((buf_a, buf_b), (buf_b, buf_a)):
                seed(wbuf)
                plsc.subcore_barrier()
                edge_pass(rbuf, wbuf)
                plsc.subcore_barrier()

        @pl.when(c == 0)
        def _():
            pltpu.sync_copy(buf_a.at[rsl], outa_hbm.at[rsl])

        @pl.when(c == 1)
        def _():
            pltpu.sync_copy(buf_a.at[rsl], outb_hbm.at[rsl])

    return prop(x0s, eb2, cnt)


def _pad_per_tile(a, pad_value):
    a = a.reshape(NUM_SUBCORES, _E_PER_TILE)
    a = jnp.pad(a, ((0, 0), (0, _E_PER_TILE_PAD - _E_PER_TILE)),
                constant_values=pad_value)
    return a.reshape(NUM_SUBCORES * _E_PER_TILE_PAD)


def kernel(x, mask, edge_index, edge_weight):
    src = edge_index[0].astype(jnp.int32)
    dst = edge_index[1].astype(jnp.int32)
    w = edge_weight.astype(jnp.float32)
    x0 = jnp.where(mask[:, None], x, 0.0)
    x0 = jnp.pad(x0, ((0, N_PAD - N_NODES), (0, 0)))
    x0s = x0.reshape(N_PAD, NUM_CORES, HALF_F).transpose(1, 0, 2)
    maski = jnp.pad(mask.astype(jnp.int32), (0, N_PAD - N_NODES))

    srcs = _pad_per_tile(src, 0).reshape(_TOTAL_BLOCKS, EDGE_BLOCK)
    dsts = _pad_per_tile(dst, 0).reshape(_TOTAL_BLOCKS, EDGE_BLOCK)
    wbits = lax.bitcast_convert_type(_pad_per_tile(w, 0.0),
                                     jnp.int32).reshape(_TOTAL_BLOCKS,
                                                        EDGE_BLOCK)
    eblk = jnp.stack([srcs, dsts, wbits], axis=1)

    eb2, cnt = _compact(eblk, maski)
    outa, outb = _propagate(x0s, eb2, cnt)
    out = jnp.stack([outa, outb], axis=1).reshape(N_PAD, D_FEAT)
    return out[:N_NODES]


# speedup vs baseline: 11.5757x; 1.0586x over previous
import dataclasses
import functools

import jax
import jax.numpy as jnp
from jax import lax
from jax.experimental import pallas as pl
from jax.experimental.pallas import tpu as pltpu
from jax.experimental.pallas import tpu_sc as plsc

N_NODES = 10000
D_FEAT = 128
N_EDGES = 320000
NUM_ITERATIONS = 10

NUM_CORES = 2
NUM_SUBCORES = 16
HALF_F = D_FEAT // NUM_CORES
EDGE_BLOCK = 128
NRING = 4

_E_PER_TILE = -(-N_EDGES // NUM_SUBCORES)
_BLOCKS_PER_TILE = NRING * (-(-_E_PER_TILE // (EDGE_BLOCK * NRING)))
_E_PER_TILE_PAD = _BLOCKS_PER_TILE * EDGE_BLOCK
_TOTAL_BLOCKS = NUM_SUBCORES * _BLOCKS_PER_TILE
_ROWS_PER_TILE = 8 * (-(-N_NODES // (8 * NUM_SUBCORES)))
N_PAD = _ROWS_PER_TILE * NUM_SUBCORES


def _compiler_params():
    cp = pltpu.CompilerParams()
    if "needs_layout_passes" in pltpu.CompilerParams.__dataclass_fields__:
        cp = dataclasses.replace(cp, needs_layout_passes=False)
    if "use_tc_tiling_on_sc" in pltpu.CompilerParams.__dataclass_fields__:
        cp = dataclasses.replace(cp, use_tc_tiling_on_sc=False)
    return cp


def _compact(eblk, maski):
    mesh = plsc.VectorSubcoreMesh(core_axis_name="c", subcore_axis_name="s")
    half_blocks = _BLOCKS_PER_TILE // 2

    @functools.partial(
        pl.kernel,
        compiler_params=_compiler_params(),
        out_type=(
            jax.ShapeDtypeStruct((NUM_SUBCORES, 3, _E_PER_TILE_PAD),
                                 jnp.int32),
            jax.ShapeDtypeStruct((NUM_SUBCORES, 16), jnp.int32),
        ),
        mesh=mesh,
        scratch_types=[
            pltpu.VMEM((N_PAD,), jnp.int32),
            pltpu.VMEM((half_blocks, 3, EDGE_BLOCK), jnp.int32),
            pltpu.VMEM((3, _E_PER_TILE_PAD), jnp.int32),
            pltpu.VMEM((16,), jnp.int32),
        ],
    )
    def comp(eblk_hbm, maski_hbm, out_hbm, cnt_hbm,
             mask_v, in_v, out_v, cw_v):
        c = lax.axis_index("c")
        s = lax.axis_index("s")

        @pl.when(c == 0)
        def _():
            pltpu.sync_copy(maski_hbm, mask_v)

            @pl.loop(0, _E_PER_TILE_PAD, step=16)
            def _(o):
                z = jnp.zeros((16,), jnp.int32)
                sl = pl.ds(o, 16)
                out_v[0, sl] = z
                out_v[1, sl] = z
                out_v[2, sl] = z

            blk0 = s * _BLOCKS_PER_TILE
            cnt = jnp.int32(0)
            for half in range(2):
                pltpu.sync_copy(
                    eblk_hbm.at[pl.ds(blk0 + half * half_blocks,
                                      half_blocks)], in_v)

                @pl.loop(0, half_blocks * (EDGE_BLOCK // 16),
                         init_carry=cnt)
                def cnt_loop(g, cnt):
                    b = g // (EDGE_BLOCK // 16)
                    sl = pl.ds((g % (EDGE_BLOCK // 16)) * 16, 16)
                    s16 = in_v[b, 0, sl]
                    d16 = in_v[b, 1, sl]
                    w16 = in_v[b, 2, sl]
                    keep = plsc.load_gather(mask_v, [d16]) == 0
                    osl = pl.ds(cnt, 16)
                    plsc.store_compressed(out_v.at[0, osl], s16, mask=keep)
                    plsc.store_compressed(out_v.at[1, osl], d16, mask=keep)
                    plsc.store_compressed(out_v.at[2, osl], w16, mask=keep)
                    return cnt + jnp.sum(jnp.where(keep, 1, 0))

                cnt = cnt_loop

            blocks = (cnt + EDGE_BLOCK - 1) // EDGE_BLOCK
            nblk = ((blocks + NRING - 1) // NRING) * NRING
            cw_v[...] = jnp.full((16,), nblk, jnp.int32)
            pltpu.sync_copy(out_v, out_hbm.at[s])
            pltpu.sync_copy(cw_v, cnt_hbm.at[s])

    return comp(eblk, maski)


def _propagate(x0s, eb2, cnt):
    mesh = plsc.VectorSubcoreMesh(core_axis_name="c", subcore_axis_name="s")

    @functools.partial(
        pl.kernel,
        compiler_params=_compiler_params(),
        out_type=(
            jax.ShapeDtypeStruct((N_PAD, HALF_F), jnp.float32),
            jax.ShapeDtypeStruct((N_PAD, HALF_F), jnp.float32),
        ),
        mesh=mesh,
        scratch_types=[
            pltpu.VMEM_SHARED((N_PAD, HALF_F), jnp.float32),
            pltpu.VMEM_SHARED((N_PAD, HALF_F), jnp.float32),
            pltpu.VMEM((NRING, 3, EDGE_BLOCK), jnp.int32),
            pltpu.VMEM((2, EDGE_BLOCK, HALF_F), jnp.float32),
            pltpu.VMEM((16,), jnp.int32),
            pltpu.SemaphoreType.DMA((NRING,)),
            pltpu.SemaphoreType.DMA((2,)),
            pltpu.SemaphoreType.DMA((2,)),
        ],
    )
    def prop(x0s_hbm, eb2_hbm, cnt_hbm, outa_hbm, outb_hbm,
             buf_a, buf_b, edges_v, rows_v, cnt_s, semE, semG, semS):
        c = lax.axis_index("c")
        s = lax.axis_index("s")
        row0 = s * _ROWS_PER_TILE
        rsl = pl.ds(row0, _ROWS_PER_TILE)

        pltpu.sync_copy(cnt_hbm.at[s], cnt_s)
        nblk = jnp.sum(jnp.where(lax.iota(jnp.int32, 16) == 0,
                                 cnt_s[...], 0), axis=0)

        def seed(wbuf):
            pltpu.sync_copy(x0s_hbm.at[c].at[rsl], wbuf.at[rsl])

        def edge_pass(rbuf, wbuf):
            def e_copy(i, p):
                return pltpu.make_async_copy(
                    eb2_hbm.at[s].at[:, pl.ds(i * EDGE_BLOCK, EDGE_BLOCK)],
                    edges_v.at[p], semE.at[p])

            def g_copy(pe, pr):
                return pltpu.make_async_copy(rbuf.at[edges_v.at[pe, 0]],
                                             rows_v.at[pr], semG.at[pr])

            def s_copy(pe, pr):
                return pltpu.make_async_copy(rows_v.at[pr],
                                             wbuf.at[edges_v.at[pe, 1]],
                                             semS.at[pr])

            def multiply(pe, pr):
                @pl.loop(0, EDGE_BLOCK, unroll=4)
                def _(e):
                    wvi = plsc.load_gather(edges_v.at[pe, 2],
                                           [jnp.full((16,), e, jnp.int32)])
                    wv = plsc.bitcast(wvi, jnp.float32)
                    for j in range(HALF_F // 16):
                        sl = pl.ds(j * 16, 16)
                        rows_v[pr, e, sl] = rows_v[pr, e, sl] * wv

            @pl.when(nblk > 0)
            def _():
                e_copy(0, 0).start()
                e_copy(1, 1).start()
                e_copy(0, 0).wait()
                g_copy(0, 0).start()

                @pl.loop(0, nblk, step=NRING)
                def _(i0):
                    for p in range(NRING):
                        i = i0 + p
                        pr = p % 2

                        @pl.when(i >= 1)
                        def _():
                            s_copy((p - 1) % NRING, (pr + 1) % 2).wait()

                        @pl.when(i + 2 < nblk)
                        def _():
                            e_copy(i + 2, (p + 2) % NRING).start()

                        @pl.when(i + 1 < nblk)
                        def _():
                            e_copy(i + 1, (p + 1) % NRING).wait()
                            g_copy((p + 1) % NRING, (pr + 1) % 2).start()

                        g_copy(p, pr).wait()
                        multiply(p, pr)
                        s_copy(p, pr).start(add=True)

                s_copy(NRING - 1, 1).wait()

        seed(buf_a)
        plsc.subcore_barrier()

        @pl.loop(0, NUM_ITERATIONS // 2)
        def _(t):
            for rbuf, wbuf in ((buf_a, buf_b), (buf_b, buf_a)):
                seed(wbuf)
                plsc.subcore_barrier()
                edge_pass(rbuf, wbuf)
                plsc.subcore_barrier()

        @pl.when(c == 0)
        def _():
            pltpu.sync_copy(buf_a.at[rsl], outa_hbm.at[rsl])

        @pl.when(c == 1)
        def _():
            pltpu.sync_copy(buf_a.at[rsl], outb_hbm.at[rsl])

    return prop(x0s, eb2, cnt)


def _pad_per_tile(a, pad_value):
    a = a.reshape(NUM_SUBCORES, _E_PER_TILE)
    a = jnp.pad(a, ((0, 0), (0, _E_PER_TILE_PAD - _E_PER_TILE)),
                constant_values=pad_value)
    return a.reshape(NUM_SUBCORES * _E_PER_TILE_PAD)


def kernel(x, mask, edge_index, edge_weight):
    src = edge_index[0].astype(jnp.int32)
    dst = edge_index[1].astype(jnp.int32)
    w = edge_weight.astype(jnp.float32)
    x0 = jnp.where(mask[:, None], x, 0.0)
    x0 = jnp.pad(x0, ((0, N_PAD - N_NODES), (0, 0)))
    x0s = x0.reshape(N_PAD, NUM_CORES, HALF_F).transpose(1, 0, 2)
    maski = jnp.pad(mask.astype(jnp.int32), (0, N_PAD - N_NODES))

    srcs = _pad_per_tile(src, 0).reshape(_TOTAL_BLOCKS, EDGE_BLOCK)
    dsts = _pad_per_tile(dst, 0).reshape(_TOTAL_BLOCKS, EDGE_BLOCK)
    wbits = lax.bitcast_convert_type(_pad_per_tile(w, 0.0),
                                     jnp.int32).reshape(_TOTAL_BLOCKS,
                                                        EDGE_BLOCK)
    eblk = jnp.stack([srcs, dsts, wbits], axis=1)

    eb2, cnt = _compact(eblk, maski)
    outa, outb = _propagate(x0s, eb2, cnt)
    out = jnp.stack([outa, outb], axis=1).reshape(N_PAD, D_FEAT)
    return out[:N_NODES]


# speedup vs baseline: 17.1717x; 1.4834x over previous
import dataclasses
import functools

import jax
import jax.numpy as jnp
from jax import lax
from jax.experimental import pallas as pl
from jax.experimental.pallas import tpu as pltpu
from jax.experimental.pallas import tpu_sc as plsc

N_NODES = 10000
D_FEAT = 128
N_EDGES = 320000
NUM_ITERATIONS = 10

NUM_CORES = 2
NUM_SUBCORES = 16
HALF_F = D_FEAT // NUM_CORES
EDGE_BLOCK = 128
NRING = 4

_E_PER_TILE = -(-N_EDGES // NUM_SUBCORES)
_BLOCKS_PER_TILE = NRING * (-(-_E_PER_TILE // (EDGE_BLOCK * NRING)))
_E_PER_TILE_PAD = _BLOCKS_PER_TILE * EDGE_BLOCK
_TOTAL_BLOCKS = NUM_SUBCORES * _BLOCKS_PER_TILE
_CAP_BLOCKS = _BLOCKS_PER_TILE + NRING
_CAP = _CAP_BLOCKS * EDGE_BLOCK
_ROWS_PER_TILE = 8 * (-(-N_NODES // (8 * NUM_SUBCORES)))
N_PAD = _ROWS_PER_TILE * NUM_SUBCORES


def _compiler_params():
    cp = pltpu.CompilerParams()
    if "needs_layout_passes" in pltpu.CompilerParams.__dataclass_fields__:
        cp = dataclasses.replace(cp, needs_layout_passes=False)
    if "use_tc_tiling_on_sc" in pltpu.CompilerParams.__dataclass_fields__:
        cp = dataclasses.replace(cp, use_tc_tiling_on_sc=False)
    return cp


def _ceil_ring(blocks):
    return ((blocks + NRING - 1) // NRING) * NRING


def _compact(eblk, maski):
    mesh = plsc.VectorSubcoreMesh(core_axis_name="c", subcore_axis_name="s")
    half_blocks = _BLOCKS_PER_TILE // 2

    @functools.partial(
        pl.kernel,
        compiler_params=_compiler_params(),
        out_type=(
            jax.ShapeDtypeStruct((NUM_SUBCORES, 3, _CAP), jnp.int32),
            jax.ShapeDtypeStruct((NUM_SUBCORES, 16), jnp.int32),
        ),
        mesh=mesh,
        scratch_types=[
            pltpu.VMEM((N_PAD,), jnp.int32),
            pltpu.VMEM((half_blocks, 3, EDGE_BLOCK), jnp.int32),
            pltpu.VMEM((3, _CAP), jnp.int32),
            pltpu.VMEM((16,), jnp.int32),
        ],
    )
    def comp(eblk_hbm, maski_hbm, out_hbm, cnt_hbm,
             mask_v, in_v, out_v, cw_v):
        c = lax.axis_index("c")
        s = lax.axis_index("s")

        @pl.when(c == 0)
        def _():
            pltpu.sync_copy(maski_hbm, mask_v)

            @pl.loop(0, _CAP, step=16)
            def _(o):
                z = jnp.zeros((16,), jnp.int32)
                sl = pl.ds(o, 16)
                out_v[0, sl] = z
                out_v[1, sl] = z
                out_v[2, sl] = z

            blk0 = s * _BLOCKS_PER_TILE
            carry = (jnp.int32(0), jnp.int32(_CAP))
            for half in range(2):
                pltpu.sync_copy(
                    eblk_hbm.at[pl.ds(blk0 + half * half_blocks,
                                      half_blocks)], in_v)

                @pl.loop(0, half_blocks * (EDGE_BLOCK // 16),
                         init_carry=carry)
                def cnt_loop(g, carry):
                    cnt_u, base_m = carry
                    b = g // (EDGE_BLOCK // 16)
                    sl = pl.ds((g % (EDGE_BLOCK // 16)) * 16, 16)
                    s16 = in_v[b, 0, sl]
                    d16 = in_v[b, 1, sl]
                    w16 = in_v[b, 2, sl]
                    dm = plsc.load_gather(mask_v, [d16]) == 0
                    sm = plsc.load_gather(mask_v, [s16]) == 0
                    keep_u = dm & sm
                    keep_m = dm & jnp.logical_not(sm)
                    usl = pl.ds(cnt_u, 16)
                    plsc.store_compressed(out_v.at[0, usl], s16, mask=keep_u)
                    plsc.store_compressed(out_v.at[1, usl], d16, mask=keep_u)
                    plsc.store_compressed(out_v.at[2, usl], w16, mask=keep_u)
                    n_m = jnp.sum(jnp.where(keep_m, 1, 0))
                    base_m = base_m - n_m
                    msl = pl.ds(base_m, 16)
                    plsc.store_compressed(out_v.at[0, msl], s16, mask=keep_m)
                    plsc.store_compressed(out_v.at[1, msl], d16, mask=keep_m)
                    plsc.store_compressed(out_v.at[2, msl], w16, mask=keep_m)
                    return (cnt_u + jnp.sum(jnp.where(keep_u, 1, 0)), base_m)

                carry = cnt_loop

            cnt_u, base_m = carry
            nblk_u = _ceil_ring((cnt_u + EDGE_BLOCK - 1) // EDGE_BLOCK)
            nblk_m = _ceil_ring(_CAP_BLOCKS - base_m // EDGE_BLOCK)
            start_m = _CAP_BLOCKS - nblk_m
            lane = lax.iota(jnp.int32, 16)
            cw_v[...] = (jnp.where(lane == 0, nblk_u, 0)
                         + jnp.where(lane == 1, start_m, 0)
                         + jnp.where(lane == 2, nblk_m, 0))
            pltpu.sync_copy(out_v, out_hbm.at[s])
            pltpu.sync_copy(cw_v, cnt_hbm.at[s])

    return comp(eblk, maski)


def _propagate(x0s, eb2, cnt):
    mesh = plsc.VectorSubcoreMesh(core_axis_name="c", subcore_axis_name="s")

    @functools.partial(
        pl.kernel,
        compiler_params=_compiler_params(),
        out_type=(
            jax.ShapeDtypeStruct((N_PAD, HALF_F), jnp.float32),
            jax.ShapeDtypeStruct((N_PAD, HALF_F), jnp.float32),
        ),
        mesh=mesh,
        scratch_types=[
            pltpu.VMEM_SHARED((N_PAD, HALF_F), jnp.float32),
            pltpu.VMEM_SHARED((N_PAD, HALF_F), jnp.float32),
            pltpu.VMEM((NRING, 3, EDGE_BLOCK), jnp.int32),
            pltpu.VMEM((2, EDGE_BLOCK, HALF_F), jnp.float32),
            pltpu.VMEM((16,), jnp.int32),
            pltpu.SemaphoreType.DMA((NRING,)),
            pltpu.SemaphoreType.DMA((2,)),
            pltpu.SemaphoreType.DMA((2,)),
        ],
    )
    def prop(x0s_hbm, eb2_hbm, cnt_hbm, outa_hbm, outb_hbm,
             buf_a, buf_b, edges_v, rows_v, cnt_s, semE, semG, semS):
        c = lax.axis_index("c")
        s = lax.axis_index("s")
        row0 = s * _ROWS_PER_TILE
        rsl = pl.ds(row0, _ROWS_PER_TILE)

        pltpu.sync_copy(cnt_hbm.at[s], cnt_s)
        lane = lax.iota(jnp.int32, 16)

        def lane_at(i):
            return jnp.sum(jnp.where(lane == i, cnt_s[...], 0), axis=0)

        nblk_u = lane_at(0)
        start_m = lane_at(1)
        nblk_m = lane_at(2)

        def seed_x0(wbuf):
            pltpu.sync_copy(x0s_hbm.at[c].at[rsl], wbuf.at[rsl])

        def save_out(buf):
            @pl.when(c == 0)
            def _():
                pltpu.sync_copy(buf.at[rsl], outa_hbm.at[rsl])

            @pl.when(c == 1)
            def _():
                pltpu.sync_copy(buf.at[rsl], outb_hbm.at[rsl])

        def seed_c(wbuf):
            @pl.when(c == 0)
            def _():
                pltpu.sync_copy(outa_hbm.at[rsl], wbuf.at[rsl])

            @pl.when(c == 1)
            def _():
                pltpu.sync_copy(outb_hbm.at[rsl], wbuf.at[rsl])

        def edge_pass(rbuf, wbuf, blk_base, nblk):
            def e_copy(i, p):
                return pltpu.make_async_copy(
                    eb2_hbm.at[s].at[:, pl.ds((blk_base + i) * EDGE_BLOCK,
                                              EDGE_BLOCK)],
                    edges_v.at[p], semE.at[p])

            def g_copy(pe, pr):
                return pltpu.make_async_copy(rbuf.at[edges_v.at[pe, 0]],
                                             rows_v.at[pr], semG.at[pr])

            def s_copy(pe, pr):
                return pltpu.make_async_copy(rows_v.at[pr],
                                             wbuf.at[edges_v.at[pe, 1]],
                                             semS.at[pr])

            def multiply(pe, pr):
                @pl.loop(0, EDGE_BLOCK, unroll=4)
                def _(e):
                    wvi = plsc.load_gather(edges_v.at[pe, 2],
                                           [jnp.full((16,), e, jnp.int32)])
                    wv = plsc.bitcast(wvi, jnp.float32)
                    for j in range(HALF_F // 16):
                        sl = pl.ds(j * 16, 16)
                        rows_v[pr, e, sl] = rows_v[pr, e, sl] * wv

            @pl.when(nblk > 0)
            def _():
                e_copy(0, 0).start()
                e_copy(1, 1).start()
                e_copy(0, 0).wait()
                g_copy(0, 0).start()

                @pl.loop(0, nblk, step=NRING)
                def _(i0):
                    for p in range(NRING):
                        i = i0 + p
                        pr = p % 2

                        @pl.when(i >= 1)
                        def _():
                            s_copy((p - 1) % NRING, (pr + 1) % 2).wait()

                        @pl.when(i + 2 < nblk)
                        def _():
                            e_copy(i + 2, (p + 2) % NRING).start()

                        @pl.when(i + 1 < nblk)
                        def _():
                            e_copy(i + 1, (p + 1) % NRING).wait()
                            g_copy((p + 1) % NRING, (pr + 1) % 2).start()

                        g_copy(p, pr).wait()
                        multiply(p, pr)
                        s_copy(p, pr).start(add=True)

                s_copy(NRING - 1, 1).wait()

        seed_x0(buf_a)
        seed_x0(buf_b)
        plsc.subcore_barrier()
        edge_pass(buf_a, buf_b, start_m, nblk_m)
        plsc.subcore_barrier()
        save_out(buf_b)
        plsc.subcore_barrier()

        edge_pass(buf_a, buf_b, 0, nblk_u)
        plsc.subcore_barrier()

        @pl.loop(0, (NUM_ITERATIONS - 2) // 2)
        def _(t):
            for rbuf, wbuf in ((buf_b, buf_a), (buf_a, buf_b)):
                seed_c(wbuf)
                plsc.subcore_barrier()
                edge_pass(rbuf, wbuf, 0, nblk_u)
                plsc.subcore_barrier()

        seed_c(buf_a)
        plsc.subcore_barrier()
        edge_pass(buf_b, buf_a, 0, nblk_u)
        plsc.subcore_barrier()
        save_out(buf_a)

    return prop(x0s, eb2, cnt)


def _pad_per_tile(a, pad_value):
    a = a.reshape(NUM_SUBCORES, _E_PER_TILE)
    a = jnp.pad(a, ((0, 0), (0, _E_PER_TILE_PAD - _E_PER_TILE)),
                constant_values=pad_value)
    return a.reshape(NUM_SUBCORES * _E_PER_TILE_PAD)


def kernel(x, mask, edge_index, edge_weight):
    src = edge_index[0].astype(jnp.int32)
    dst = edge_index[1].astype(jnp.int32)
    w = edge_weight.astype(jnp.float32)
    x0 = jnp.where(mask[:, None], x, 0.0)
    x0 = jnp.pad(x0, ((0, N_PAD - N_NODES), (0, 0)))
    x0s = x0.reshape(N_PAD, NUM_CORES, HALF_F).transpose(1, 0, 2)
    maski = jnp.pad(mask.astype(jnp.int32), (0, N_PAD - N_NODES))

    srcs = _pad_per_tile(src, 0).reshape(_TOTAL_BLOCKS, EDGE_BLOCK)
    dsts = _pad_per_tile(dst, 0).reshape(_TOTAL_BLOCKS, EDGE_BLOCK)
    wbits = lax.bitcast_convert_type(_pad_per_tile(w, 0.0),
                                     jnp.int32).reshape(_TOTAL_BLOCKS,
                                                        EDGE_BLOCK)
    eblk = jnp.stack([srcs, dsts, wbits], axis=1)

    eb2, cnt = _compact(eblk, maski)
    outa, outb = _propagate(x0s, eb2, cnt)
    out = jnp.stack([outa, outb], axis=1).reshape(N_PAD, D_FEAT)
    return out[:N_NODES]
